# baseline (device time: 19744 ns/iter reference)
import jax
import jax.numpy as jnp
from jax import lax
from jax.experimental import pallas as pl
from jax.experimental.pallas import tpu as pltpu

N_DEV = 16
N_IDX = 512
ROWS_PER = 2048
D = 256
N_RAILS = 4
Q = N_IDX // N_RAILS
HQ = Q // 2
N_SLOTS = 6

BASE_MASKS = (1, 2, 4, 8)
MASKS = tuple(
    tuple(BASE_MASKS[(s + r) % 4] for s in range(4)) + (BASE_MASKS[r],)
    for r in range(N_RAILS)
)
HBIT = {1: 0, 2: 1, 4: 2, 8: 3}


def kernel(table, idx):
    idx2 = idx.reshape(N_IDX, 1)

    def body(table_ref, idx_ref, out_ref, init_ref, piece_ref, recv_ref,
             send_sems, recv_sems):
        my = lax.axis_index("i")

        barrier_sem = pltpu.get_barrier_semaphore()
        for m in BASE_MASKS:
            pl.semaphore_signal(
                barrier_sem,
                inc=1,
                device_id=(my ^ m,),
                device_id_type=pl.DeviceIdType.MESH,
            )

        table_bf16 = table_ref[:, :].astype(jnp.bfloat16)

        def offs(r):
            bit = HBIT[MASKS[r][0]]
            keep = ((my >> bit) & 1) * HQ
            return keep, HQ - keep

        def partial_quarter(r):
            local = idx_ref[pl.ds(r * Q, Q), :] - my * ROWS_PER
            cols = lax.broadcasted_iota(jnp.int16, (Q, ROWS_PER), 1)
            onehot = (cols == local.astype(jnp.int16)).astype(jnp.bfloat16)
            acc = lax.dot_general(
                onehot,
                table_bf16,
                (((1,), (0,)), ((), ())),
                preferred_element_type=jnp.float32,
            )
            init_ref[r] = acc.astype(jnp.bfloat16)

        def start(src_ref, s, rail):
            rdma = pltpu.make_async_remote_copy(
                src_ref=src_ref,
                dst_ref=recv_ref.at[s, rail],
                send_sem=send_sems.at[s, rail],
                recv_sem=recv_sems.at[s, rail],
                device_id=(my ^ MASKS[rail][s],),
                device_id_type=pl.DeviceIdType.MESH,
            )
            rdma.start()
            return rdma

        sends = {}
        partial_quarter(0)
        pl.semaphore_wait(barrier_sem, 4)
        keep0, send0 = offs(0)
        sends[0, 0] = start(init_ref.at[0, pl.ds(send0, HQ), :], 0, 0)
        for r in range(1, N_RAILS):
            partial_quarter(r)
            keep, send = offs(r)
            sends[0, r] = start(init_ref.at[r, pl.ds(send, HQ), :], 0, r)

        for r in range(N_RAILS):
            sends[0, r].wait_recv()
            keep, _ = offs(r)
            piece_ref[r, 0] = (
                init_ref[r, pl.ds(keep, HQ), :] + recv_ref[0, r]
            )
            sends[1, r] = start(piece_ref.at[r, 0], 1, r)

        for s in range(1, 4):
            for r in range(N_RAILS):
                sends[s, r].wait_recv()
                if s >= 2:
                    sends[s - 1, r].wait_send()
                piece_ref[r, s % 2] = (
                    piece_ref[r, (s - 1) % 2] + recv_ref[s, r]
                )
                sends[s + 1, r] = start(piece_ref.at[r, s % 2], s + 1, r)

        for r in range(N_RAILS):
            sends[4, r].wait_recv()
            sends[3, r].wait_send()
            keep, send = offs(r)
            base = r * Q
            out_ref[pl.ds(base + send, HQ), :] = recv_ref[4, r]
            out_ref[pl.ds(base + keep, HQ), :] = piece_ref[r, 1]

        for r in range(N_RAILS):
            sends[0, r].wait_send()
            sends[4, r].wait_send()

    return pl.pallas_call(
        body,
        out_shape=jax.ShapeDtypeStruct((N_IDX, D), jnp.bfloat16),
        in_specs=[
            pl.BlockSpec(memory_space=pltpu.VMEM),
            pl.BlockSpec(memory_space=pltpu.VMEM),
        ],
        out_specs=pl.BlockSpec(memory_space=pltpu.VMEM),
        scratch_shapes=[
            pltpu.VMEM((N_RAILS, Q, D), jnp.bfloat16),
            pltpu.VMEM((N_RAILS, 2, HQ, D), jnp.bfloat16),
            pltpu.VMEM((5, N_RAILS, HQ, D), jnp.bfloat16),
            pltpu.SemaphoreType.DMA((5, N_RAILS)),
            pltpu.SemaphoreType.DMA((5, N_RAILS)),
        ],
        compiler_params=pltpu.CompilerParams(collective_id=0),
    )(table, idx2)
